# baseline (speedup 1.0000x reference)
import jax
import jax.numpy as jnp
from jax import lax
from jax.experimental import pallas as pl
from jax.experimental.pallas import tpu as pltpu

BLOCK_P = 12500
NBUF = 2
NSPLIT = 4

BIG = 2**31 - 1
INF = float("inf")


def _colmin(x):
    return jnp.min(x, axis=1, keepdims=True)


def _topk_body(q_ref, kv_hbm, dists_ref, idx_ref, buf_ref, sems):
    j = pl.program_id(0)
    nsteps = pl.num_programs(0)
    chunk = BLOCK_P // NSPLIT

    def block_copy(jj, slot, s):
        return pltpu.make_async_copy(
            kv_hbm.at[pl.ds(jj * BLOCK_P + s * chunk, chunk), :],
            buf_ref.at[slot, pl.ds(s * chunk, chunk), :],
            sems.at[slot, s])

    @pl.when(j == 0)
    def _prime():
        for s in range(NSPLIT):
            block_copy(0, 0, s).start()

    @pl.when(j + 1 < nsteps)
    def _prefetch():
        nslot = lax.rem(j + 1, NBUF)
        for s in range(NSPLIT):
            block_copy(j + 1, nslot, s).start()

    slot = lax.rem(j, NBUF)
    for s in range(NSPLIT):
        block_copy(j, slot, s).wait()

    k2 = buf_ref[slot]
    ka = k2[:, 0:64]
    kb = k2[:, 64:128]
    q = q_ref[...]
    q_sq = jnp.sum(q * q, axis=1, keepdims=True)

    def half_dists(kh):
        k_sq = jnp.sum(kh * kh, axis=1)
        qk = lax.dot_general(q, kh, (((1,), (1,)), ((), ())),
                             preferred_element_type=jnp.float32)
        return q_sq + k_sq[None, :] - 2.0 * qk

    d_e = half_dists(ka)
    d_o = half_dists(kb)
    iota = lax.broadcasted_iota(jnp.int32, d_e.shape, 1)
    base = j * (2 * BLOCK_P)

    m1 = jnp.minimum(_colmin(d_e), _colmin(d_o))
    ce = _colmin(jnp.where(d_e == m1, iota, BIG))
    co = _colmin(jnp.where(d_o == m1, iota, BIG))
    i1 = jnp.minimum(
        jnp.where(ce == BIG, BIG, base + 2 * ce),
        jnp.where(co == BIG, BIG, base + 2 * co + 1))

    rel = i1 - base
    excl_e = jnp.where(rel % 2 == 0, rel // 2, jnp.int32(-1))
    excl_o = jnp.where(rel % 2 == 1, rel // 2, jnp.int32(-1))
    d2_e = jnp.where(iota == excl_e, INF, d_e)
    d2_o = jnp.where(iota == excl_o, INF, d_o)
    m2 = jnp.minimum(_colmin(d2_e), _colmin(d2_o))
    ce2 = _colmin(jnp.where(d2_e == m2, iota, BIG))
    co2 = _colmin(jnp.where(d2_o == m2, iota, BIG))
    i2 = jnp.minimum(
        jnp.where(ce2 == BIG, BIG, base + 2 * ce2),
        jnp.where(co2 == BIG, BIG, base + 2 * co2 + 1))

    @pl.when(j == 0)
    def _init():
        dists_ref[...] = jnp.concatenate([m1, m2], axis=1)
        idx_ref[...] = jnp.concatenate([i1, i2], axis=1)

    @pl.when(j > 0)
    def _merge():
        a1 = dists_ref[:, 0:1]
        a2 = dists_ref[:, 1:2]
        ia1 = idx_ref[:, 0:1]
        ia2 = idx_ref[:, 1:2]
        keep1 = a1 <= m1
        n1 = jnp.where(keep1, a1, m1)
        ni1 = jnp.where(keep1, ia1, i1)
        n2 = jnp.where(keep1,
                       jnp.where(a2 <= m1, a2, m1),
                       jnp.where(a1 <= m2, a1, m2))
        ni2 = jnp.where(keep1,
                        jnp.where(a2 <= m1, ia2, i1),
                        jnp.where(a1 <= m2, ia1, i2))
        dists_ref[...] = jnp.concatenate([n1, n2], axis=1)
        idx_ref[...] = jnp.concatenate([ni1, ni2], axis=1)


def _topk2(queries, kview, interpret=False):
    pn, two_d = kview.shape
    q = queries.shape[0]
    nsteps = pn // BLOCK_P
    return pl.pallas_call(
        _topk_body,
        grid=(nsteps,),
        in_specs=[
            pl.BlockSpec((q, two_d // 2), lambda j: (0, 0)),
            pl.BlockSpec(memory_space=pltpu.MemorySpace.HBM),
        ],
        out_specs=[
            pl.BlockSpec((q, 2), lambda j: (0, 0)),
            pl.BlockSpec((q, 2), lambda j: (0, 0)),
        ],
        out_shape=[
            jax.ShapeDtypeStruct((q, 2), jnp.float32),
            jax.ShapeDtypeStruct((q, 2), jnp.int32),
        ],
        scratch_shapes=[
            pltpu.VMEM((NBUF, BLOCK_P, 128), jnp.float32),
            pltpu.SemaphoreType.DMA((NBUF, NSPLIT)),
        ],
        interpret=interpret,
    )(queries, kview)


def _gather_rows(kview, idx0):
    q = idx0.shape[0]
    d = kview.shape[1] // 2

    def body(idx_ref, idxv_ref, kv_ref, out_ref, rows_ref, sem):
        def row_copy(i):
            p = idx_ref[i] // 2
            return pltpu.make_async_copy(
                kv_ref.at[pl.ds(p, 1), :],
                rows_ref.at[pl.ds(i, 1), :], sem)

        def start(i, _):
            row_copy(i).start()
            return 0

        def drain(i, _):
            row_copy(i).wait()
            return 0

        lax.fori_loop(0, q, start, 0)
        lax.fori_loop(0, q, drain, 0)
        rows = rows_ref[...]
        odd = (idxv_ref[...] % 2) == 1
        out_ref[...] = jnp.where(odd, rows[:, d:], rows[:, :d])

    return pl.pallas_call(
        body,
        in_specs=[
            pl.BlockSpec(memory_space=pltpu.MemorySpace.SMEM),
            pl.BlockSpec((q, 1), lambda: (0, 0)),
            pl.BlockSpec(memory_space=pltpu.MemorySpace.HBM),
        ],
        out_specs=pl.BlockSpec(memory_space=pltpu.MemorySpace.VMEM),
        out_shape=jax.ShapeDtypeStruct((q, d), jnp.float32),
        scratch_shapes=[
            pltpu.VMEM((q, 2 * d), jnp.float32),
            pltpu.SemaphoreType.DMA,
        ],
    )(idx0, idx0[:, None], kview)


def kernel(queries, keys):
    kn, d = keys.shape
    kview = jnp.reshape(keys, (kn // 2, 2 * d))
    dists, idx = _topk2(queries, kview)
    retrieved = _gather_rows(kview, idx[:, 0])
    return (dists, idx, retrieved)


# speedup vs baseline: 1.6844x; 1.6844x over previous
import jax
import jax.numpy as jnp
from jax import lax
from jax.experimental import pallas as pl
from jax.experimental.pallas import tpu as pltpu

BLOCK_K = 25000
NBUF = 2
NSPLIT = 4

BIG = 2**31 - 1
INF = float("inf")


def _colmin(x):
    return jnp.min(x, axis=1, keepdims=True)


def _topk_body(q_ref, k_hbm, dists_ref, idx_ref, buf_ref, sems):
    j = pl.program_id(0)
    nsteps = pl.num_programs(0)
    chunk = BLOCK_K // NSPLIT

    def block_copy(jj, slot, s):
        return pltpu.make_async_copy(
            k_hbm.at[pl.ds(jj * BLOCK_K + s * chunk, chunk), :],
            buf_ref.at[slot, pl.ds(s * chunk, chunk), :],
            sems.at[slot, s])

    @pl.when(j == 0)
    def _prime():
        for s in range(NSPLIT):
            block_copy(0, 0, s).start()

    @pl.when(j + 1 < nsteps)
    def _prefetch():
        nslot = lax.rem(j + 1, NBUF)
        for s in range(NSPLIT):
            block_copy(j + 1, nslot, s).start()

    slot = lax.rem(j, NBUF)
    for s in range(NSPLIT):
        block_copy(j, slot, s).wait()

    k = buf_ref[slot]
    q = q_ref[...]
    q_sq = jnp.sum(q * q, axis=1, keepdims=True)
    k_sq = jnp.sum(k * k, axis=1)
    qk2 = lax.dot_general(q + q, k, (((1,), (1,)), ((), ())),
                          preferred_element_type=jnp.float32)
    d = (q_sq + k_sq[None, :]) - qk2

    iota = lax.broadcasted_iota(jnp.int32, d.shape, 1)
    m1 = _colmin(d)
    c1 = _colmin(jnp.where(d == m1, iota, BIG))
    d2 = jnp.where(iota == c1, INF, d)
    m2 = _colmin(d2)
    c2 = _colmin(jnp.where(d2 == m2, iota, BIG))
    base = j * BLOCK_K
    i1 = base + c1
    i2 = base + c2

    @pl.when(j == 0)
    def _init():
        dists_ref[...] = jnp.concatenate([m1, m2], axis=1)
        idx_ref[...] = jnp.concatenate([i1, i2], axis=1)

    @pl.when(j > 0)
    def _merge():
        a1 = dists_ref[:, 0:1]
        a2 = dists_ref[:, 1:2]
        ia1 = idx_ref[:, 0:1]
        ia2 = idx_ref[:, 1:2]
        keep1 = a1 <= m1
        n1 = jnp.where(keep1, a1, m1)
        ni1 = jnp.where(keep1, ia1, i1)
        n2 = jnp.where(keep1,
                       jnp.where(a2 <= m1, a2, m1),
                       jnp.where(a1 <= m2, a1, m2))
        ni2 = jnp.where(keep1,
                        jnp.where(a2 <= m1, ia2, i1),
                        jnp.where(a1 <= m2, ia1, i2))
        dists_ref[...] = jnp.concatenate([n1, n2], axis=1)
        idx_ref[...] = jnp.concatenate([ni1, ni2], axis=1)


def _topk2(queries, keys, interpret=False):
    kn, d = keys.shape
    q = queries.shape[0]
    nsteps = kn // BLOCK_K
    return pl.pallas_call(
        _topk_body,
        grid=(nsteps,),
        in_specs=[
            pl.BlockSpec((q, d), lambda j: (0, 0)),
            pl.BlockSpec(memory_space=pltpu.MemorySpace.HBM),
        ],
        out_specs=[
            pl.BlockSpec((q, 2), lambda j: (0, 0)),
            pl.BlockSpec((q, 2), lambda j: (0, 0)),
        ],
        out_shape=[
            jax.ShapeDtypeStruct((q, 2), jnp.float32),
            jax.ShapeDtypeStruct((q, 2), jnp.int32),
        ],
        scratch_shapes=[
            pltpu.VMEM((NBUF, BLOCK_K, 64), jnp.float32),
            pltpu.SemaphoreType.DMA((NBUF, NSPLIT)),
        ],
        interpret=interpret,
    )(queries, keys)


def _gather_rows(keys, idx0):
    q = idx0.shape[0]
    d = keys.shape[1]

    def body(idx_ref, keys_ref, out_ref, sem):
        def row_copy(i):
            return pltpu.make_async_copy(
                keys_ref.at[pl.ds(idx_ref[i], 1), :],
                out_ref.at[pl.ds(i, 1), :], sem)

        def start(i, _):
            row_copy(i).start()
            return 0

        def drain(i, _):
            row_copy(i).wait()
            return 0

        lax.fori_loop(0, q, start, 0)
        lax.fori_loop(0, q, drain, 0)

    return pl.pallas_call(
        body,
        in_specs=[
            pl.BlockSpec(memory_space=pltpu.MemorySpace.SMEM),
            pl.BlockSpec(memory_space=pltpu.MemorySpace.HBM),
        ],
        out_specs=pl.BlockSpec(memory_space=pltpu.MemorySpace.VMEM),
        out_shape=jax.ShapeDtypeStruct((q, d), jnp.float32),
        scratch_shapes=[pltpu.SemaphoreType.DMA],
    )(idx0, keys)


def kernel(queries, keys):
    dists, idx = _topk2(queries, keys)
    retrieved = _gather_rows(keys, idx[:, 0])
    return (dists, idx, retrieved)


# speedup vs baseline: 1.7150x; 1.0181x over previous
import jax
import jax.numpy as jnp
from jax import lax
from jax.experimental import pallas as pl
from jax.experimental.pallas import tpu as pltpu

BLOCK_K = 50000
NBUF = 2
NSPLIT = 4

BIG = 2**31 - 1
INF = float("inf")


def _colmin(x):
    return jnp.min(x, axis=1, keepdims=True)


def _topk_body(q_ref, k_hbm, dists_ref, idx_ref, buf_ref, sems):
    j = pl.program_id(0)
    nsteps = pl.num_programs(0)
    chunk = BLOCK_K // NSPLIT

    def block_copy(jj, slot, s):
        return pltpu.make_async_copy(
            k_hbm.at[pl.ds(jj * BLOCK_K + s * chunk, chunk), :],
            buf_ref.at[slot, pl.ds(s * chunk, chunk), :],
            sems.at[slot, s])

    @pl.when(j == 0)
    def _prime():
        for s in range(NSPLIT):
            block_copy(0, 0, s).start()

    @pl.when(j + 1 < nsteps)
    def _prefetch():
        nslot = lax.rem(j + 1, NBUF)
        for s in range(NSPLIT):
            block_copy(j + 1, nslot, s).start()

    slot = lax.rem(j, NBUF)
    for s in range(NSPLIT):
        block_copy(j, slot, s).wait()

    k = buf_ref[slot]
    q = q_ref[...]
    q_sq = jnp.sum(q * q, axis=1, keepdims=True)
    k_sq = jnp.sum(k * k, axis=1)
    qk2 = lax.dot_general(q + q, k, (((1,), (1,)), ((), ())),
                          preferred_element_type=jnp.float32)
    d = (q_sq + k_sq[None, :]) - qk2

    iota = lax.broadcasted_iota(jnp.int32, d.shape, 1)
    m1 = _colmin(d)
    c1 = _colmin(jnp.where(d == m1, iota, BIG))
    d2 = jnp.where(iota == c1, INF, d)
    m2 = _colmin(d2)
    c2 = _colmin(jnp.where(d2 == m2, iota, BIG))
    base = j * BLOCK_K
    i1 = base + c1
    i2 = base + c2

    @pl.when(j == 0)
    def _init():
        dists_ref[...] = jnp.concatenate([m1, m2], axis=1)
        idx_ref[...] = jnp.concatenate([i1, i2], axis=1)

    @pl.when(j > 0)
    def _merge():
        a1 = dists_ref[:, 0:1]
        a2 = dists_ref[:, 1:2]
        ia1 = idx_ref[:, 0:1]
        ia2 = idx_ref[:, 1:2]
        keep1 = a1 <= m1
        n1 = jnp.where(keep1, a1, m1)
        ni1 = jnp.where(keep1, ia1, i1)
        n2 = jnp.where(keep1,
                       jnp.where(a2 <= m1, a2, m1),
                       jnp.where(a1 <= m2, a1, m2))
        ni2 = jnp.where(keep1,
                        jnp.where(a2 <= m1, ia2, i1),
                        jnp.where(a1 <= m2, ia1, i2))
        dists_ref[...] = jnp.concatenate([n1, n2], axis=1)
        idx_ref[...] = jnp.concatenate([ni1, ni2], axis=1)


def _topk2(queries, keys, interpret=False):
    kn, d = keys.shape
    q = queries.shape[0]
    nsteps = kn // BLOCK_K
    return pl.pallas_call(
        _topk_body,
        grid=(nsteps,),
        in_specs=[
            pl.BlockSpec((q, d), lambda j: (0, 0)),
            pl.BlockSpec(memory_space=pltpu.MemorySpace.HBM),
        ],
        out_specs=[
            pl.BlockSpec((q, 2), lambda j: (0, 0)),
            pl.BlockSpec((q, 2), lambda j: (0, 0)),
        ],
        out_shape=[
            jax.ShapeDtypeStruct((q, 2), jnp.float32),
            jax.ShapeDtypeStruct((q, 2), jnp.int32),
        ],
        scratch_shapes=[
            pltpu.VMEM((NBUF, BLOCK_K, 64), jnp.float32),
            pltpu.SemaphoreType.DMA((NBUF, NSPLIT)),
        ],
        interpret=interpret,
    )(queries, keys)


def _gather_rows(keys, idx0):
    q = idx0.shape[0]
    d = keys.shape[1]

    def body(idx_ref, keys_ref, out_ref, sem):
        def row_copy(i):
            return pltpu.make_async_copy(
                keys_ref.at[pl.ds(idx_ref[i], 1), :],
                out_ref.at[pl.ds(i, 1), :], sem)

        def start(i, _):
            row_copy(i).start()
            return 0

        def drain(i, _):
            row_copy(i).wait()
            return 0

        lax.fori_loop(0, q, start, 0)
        lax.fori_loop(0, q, drain, 0)

    return pl.pallas_call(
        body,
        in_specs=[
            pl.BlockSpec(memory_space=pltpu.MemorySpace.SMEM),
            pl.BlockSpec(memory_space=pltpu.MemorySpace.HBM),
        ],
        out_specs=pl.BlockSpec(memory_space=pltpu.MemorySpace.VMEM),
        out_shape=jax.ShapeDtypeStruct((q, d), jnp.float32),
        scratch_shapes=[pltpu.SemaphoreType.DMA],
    )(idx0, keys)


def kernel(queries, keys):
    dists, idx = _topk2(queries, keys)
    retrieved = _gather_rows(keys, idx[:, 0])
    return (dists, idx, retrieved)


# speedup vs baseline: 1.7951x; 1.0467x over previous
import jax
import jax.numpy as jnp
from jax import lax
from jax.experimental import pallas as pl
from jax.experimental.pallas import tpu as pltpu

BLOCK_K = 50000
NBUF = 2
NSPLIT = 4

BIG = 2**31 - 1
INF = float("inf")


def _colmin(x):
    return jnp.min(x, axis=1, keepdims=True)


def _topk_body(q_ref, k_hbm, dists_ref, idx_ref, buf_ref, sems):
    j = pl.program_id(0)
    nsteps = pl.num_programs(0)
    chunk = BLOCK_K // NSPLIT

    def block_copy(jj, slot, s):
        return pltpu.make_async_copy(
            k_hbm.at[pl.ds(jj * BLOCK_K + s * chunk, chunk), :],
            buf_ref.at[slot, pl.ds(s * chunk, chunk), :],
            sems.at[slot, s])

    @pl.when(j == 0)
    def _prime():
        for s in range(NSPLIT):
            block_copy(0, 0, s).start()

    @pl.when(j + 1 < nsteps)
    def _prefetch():
        nslot = lax.rem(j + 1, NBUF)
        for s in range(NSPLIT):
            block_copy(j + 1, nslot, s).start()

    slot = lax.rem(j, NBUF)
    for s in range(NSPLIT):
        block_copy(j, slot, s).wait()

    k = buf_ref[slot]
    q = q_ref[...]
    q_sq = jnp.sum(q * q, axis=1, keepdims=True)
    k_sq = jnp.sum(k * k, axis=1)
    qk2 = lax.dot_general(q + q, k, (((1,), (1,)), ((), ())),
                          preferred_element_type=jnp.float32)
    d = (q_sq + k_sq[None, :]) - qk2

    iota = lax.broadcasted_iota(jnp.int32, d.shape, 1)
    c1 = jnp.argmin(d, axis=1, keepdims=True).astype(jnp.int32)
    m1 = _colmin(d)
    d2 = jnp.where(iota == c1, INF, d)
    c2 = jnp.argmin(d2, axis=1, keepdims=True).astype(jnp.int32)
    m2 = _colmin(d2)
    base = j * BLOCK_K
    i1 = base + c1
    i2 = base + c2

    @pl.when(j == 0)
    def _init():
        dists_ref[...] = jnp.concatenate([m1, m2], axis=1)
        idx_ref[...] = jnp.concatenate([i1, i2], axis=1)

    @pl.when(j > 0)
    def _merge():
        a1 = dists_ref[:, 0:1]
        a2 = dists_ref[:, 1:2]
        ia1 = idx_ref[:, 0:1]
        ia2 = idx_ref[:, 1:2]
        keep1 = a1 <= m1
        n1 = jnp.where(keep1, a1, m1)
        ni1 = jnp.where(keep1, ia1, i1)
        n2 = jnp.where(keep1,
                       jnp.where(a2 <= m1, a2, m1),
                       jnp.where(a1 <= m2, a1, m2))
        ni2 = jnp.where(keep1,
                        jnp.where(a2 <= m1, ia2, i1),
                        jnp.where(a1 <= m2, ia1, i2))
        dists_ref[...] = jnp.concatenate([n1, n2], axis=1)
        idx_ref[...] = jnp.concatenate([ni1, ni2], axis=1)


def _topk2(queries, keys, interpret=False):
    kn, d = keys.shape
    q = queries.shape[0]
    nsteps = kn // BLOCK_K
    return pl.pallas_call(
        _topk_body,
        grid=(nsteps,),
        in_specs=[
            pl.BlockSpec((q, d), lambda j: (0, 0)),
            pl.BlockSpec(memory_space=pltpu.MemorySpace.HBM),
        ],
        out_specs=[
            pl.BlockSpec((q, 2), lambda j: (0, 0)),
            pl.BlockSpec((q, 2), lambda j: (0, 0)),
        ],
        out_shape=[
            jax.ShapeDtypeStruct((q, 2), jnp.float32),
            jax.ShapeDtypeStruct((q, 2), jnp.int32),
        ],
        scratch_shapes=[
            pltpu.VMEM((NBUF, BLOCK_K, 64), jnp.float32),
            pltpu.SemaphoreType.DMA((NBUF, NSPLIT)),
        ],
        interpret=interpret,
    )(queries, keys)


def _gather_rows(keys, idx0):
    q = idx0.shape[0]
    d = keys.shape[1]

    def body(idx_ref, keys_ref, out_ref, sem):
        def row_copy(i):
            return pltpu.make_async_copy(
                keys_ref.at[pl.ds(idx_ref[i], 1), :],
                out_ref.at[pl.ds(i, 1), :], sem)

        def start(i, _):
            row_copy(i).start()
            return 0

        def drain(i, _):
            row_copy(i).wait()
            return 0

        lax.fori_loop(0, q, start, 0)
        lax.fori_loop(0, q, drain, 0)

    return pl.pallas_call(
        body,
        in_specs=[
            pl.BlockSpec(memory_space=pltpu.MemorySpace.SMEM),
            pl.BlockSpec(memory_space=pltpu.MemorySpace.HBM),
        ],
        out_specs=pl.BlockSpec(memory_space=pltpu.MemorySpace.VMEM),
        out_shape=jax.ShapeDtypeStruct((q, d), jnp.float32),
        scratch_shapes=[pltpu.SemaphoreType.DMA],
    )(idx0, keys)


def kernel(queries, keys):
    dists, idx = _topk2(queries, keys)
    retrieved = _gather_rows(keys, idx[:, 0])
    return (dists, idx, retrieved)
